# baseline (speedup 1.0000x reference)
import jax
import jax.numpy as jnp
import numpy as np
from jax.experimental import pallas as pl
from jax.experimental.pallas import tpu as pltpu

T = 4096
D = 1024
OUT = 1024
R = 64
E = 16
K = 2
TB = 512

_S1 = np.zeros((E, E * R), np.float32)
for _e in range(E):
    _S1[_e, _e * R:(_e + 1) * R] = 1.0
_S2 = np.tile(np.eye(R, dtype=np.float32), (1, E))


def _fused_body(x_ref, wb_ref, b_ref, a_ref, bexp_ref, wr_ref, s1_ref, s2_ref,
                o_ref, wbs_ref, b2s_ref):
    @pl.when(pl.program_id(0) == 0)
    def _prep():
        wbs_ref[...] = wb_ref[...].astype(jnp.bfloat16)
        for e in range(E):
            b2s_ref[e * R:(e + 1) * R, :] = (
                bexp_ref[e].T.astype(jnp.bfloat16))

    x = x_ref[...]
    xb = x.astype(jnp.bfloat16)
    dn = (((1,), (1,)), ((), ()))
    h = jax.lax.dot_general(x, a_ref[...], dn,
                            preferred_element_type=jnp.float32)
    logits = jax.lax.dot_general(x, wr_ref[...], dn,
                                 preferred_element_type=jnp.float32)
    w = jax.nn.softmax(logits, axis=-1)
    eids = jax.lax.broadcasted_iota(jnp.int32, w.shape, 1)
    i1 = jnp.argmax(w, axis=-1)
    w1 = jnp.max(w, axis=-1)
    masked = jnp.where(eids == i1[:, None], -jnp.inf, w)
    i2 = jnp.argmax(masked, axis=-1)
    w2 = jnp.max(masked, axis=-1)
    s = w1 + w2 + 1e-6
    w_eff = (jnp.where(eids == i1[:, None], w1[:, None], 0.0)
             + jnp.where(eids == i2[:, None], w2[:, None], 0.0)) / s[:, None]
    w_rep = jnp.dot(w_eff, s1_ref[...], preferred_element_type=jnp.float32)
    h_tile = jnp.dot(h, s2_ref[...], preferred_element_type=jnp.float32)
    p = (w_rep * h_tile).astype(jnp.bfloat16)
    acc = jax.lax.dot_general(xb, wbs_ref[...], dn,
                              preferred_element_type=jnp.float32)
    acc = acc + jnp.dot(p, b2s_ref[...], preferred_element_type=jnp.float32)
    o_ref[...] = acc + b_ref[...]


def kernel(x, W_base, b_base, A, B, Wr):
    b2d = b_base.reshape(1, OUT)
    s1 = jnp.asarray(_S1)
    s2 = jnp.asarray(_S2)
    grid = (T // TB,)
    return pl.pallas_call(
        _fused_body,
        grid=grid,
        in_specs=[
            pl.BlockSpec((TB, D), lambda i: (i, 0)),
            pl.BlockSpec((OUT, D), lambda i: (0, 0)),
            pl.BlockSpec((1, OUT), lambda i: (0, 0)),
            pl.BlockSpec((R, D), lambda i: (0, 0)),
            pl.BlockSpec((E, OUT, R), lambda i: (0, 0, 0)),
            pl.BlockSpec((E, D), lambda i: (0, 0)),
            pl.BlockSpec((E, E * R), lambda i: (0, 0)),
            pl.BlockSpec((R, E * R), lambda i: (0, 0)),
        ],
        out_specs=pl.BlockSpec((TB, OUT), lambda i: (i, 0)),
        out_shape=jax.ShapeDtypeStruct((T, OUT), jnp.float32),
        scratch_shapes=[
            pltpu.VMEM((OUT, D), jnp.bfloat16),
            pltpu.VMEM((E * R, OUT), jnp.bfloat16),
        ],
    )(x, W_base, b2d, A, B, Wr, s1, s2)


# speedup vs baseline: 1.1236x; 1.1236x over previous
import jax
import jax.numpy as jnp
import numpy as np
from jax.experimental import pallas as pl

T = 4096
D = 1024
OUT = 1024
R = 64
E = 16
K = 2
TB = 512

_S1 = np.zeros((E, E * R), np.float32)
for _e in range(E):
    _S1[_e, _e * R:(_e + 1) * R] = 1.0
_S2 = np.tile(np.eye(R, dtype=np.float32), (1, E))


def _fused_body(x_ref, wb_ref, b_ref, a_ref, b2_ref, wr_ref, s1_ref, s2_ref,
                o_ref):
    x = x_ref[...]
    xb = x.astype(jnp.bfloat16)
    dn = (((1,), (1,)), ((), ()))
    h = jax.lax.dot_general(x, a_ref[...], dn,
                            preferred_element_type=jnp.float32)
    logits = jax.lax.dot_general(x, wr_ref[...], dn,
                                 preferred_element_type=jnp.float32)
    w = jax.nn.softmax(logits, axis=-1)
    eids = jax.lax.broadcasted_iota(jnp.int32, w.shape, 1)
    i1 = jnp.argmax(w, axis=-1)
    w1 = jnp.max(w, axis=-1)
    masked = jnp.where(eids == i1[:, None], -jnp.inf, w)
    i2 = jnp.argmax(masked, axis=-1)
    w2 = jnp.max(masked, axis=-1)
    s = w1 + w2 + 1e-6
    w_eff = (jnp.where(eids == i1[:, None], w1[:, None], 0.0)
             + jnp.where(eids == i2[:, None], w2[:, None], 0.0)) / s[:, None]
    w_rep = jnp.dot(w_eff, s1_ref[...], preferred_element_type=jnp.float32)
    h_tile = jnp.dot(h, s2_ref[...], preferred_element_type=jnp.float32)
    p = (w_rep * h_tile).astype(jnp.bfloat16)
    acc = jax.lax.dot_general(xb, wb_ref[...], dn,
                              preferred_element_type=jnp.float32)
    acc = acc + jnp.dot(p, b2_ref[...], preferred_element_type=jnp.float32)
    o_ref[...] = acc + b_ref[...]


def kernel(x, W_base, b_base, A, B, Wr):
    b2d = b_base.reshape(1, OUT)
    wb = W_base.astype(jnp.bfloat16)
    b2 = B.transpose(0, 2, 1).reshape(E * R, OUT).astype(jnp.bfloat16)
    s1 = jnp.asarray(_S1)
    s2 = jnp.asarray(_S2)
    grid = (T // TB,)
    return pl.pallas_call(
        _fused_body,
        grid=grid,
        in_specs=[
            pl.BlockSpec((TB, D), lambda i: (i, 0)),
            pl.BlockSpec((OUT, D), lambda i: (0, 0)),
            pl.BlockSpec((1, OUT), lambda i: (0, 0)),
            pl.BlockSpec((R, D), lambda i: (0, 0)),
            pl.BlockSpec((E * R, OUT), lambda i: (0, 0)),
            pl.BlockSpec((E, D), lambda i: (0, 0)),
            pl.BlockSpec((E, E * R), lambda i: (0, 0)),
            pl.BlockSpec((R, E * R), lambda i: (0, 0)),
        ],
        out_specs=pl.BlockSpec((TB, OUT), lambda i: (i, 0)),
        out_shape=jax.ShapeDtypeStruct((T, OUT), jnp.float32),
    )(x, wb, b2d, A, b2, Wr, s1, s2)
